# baseline (device time: 704287 ns/iter reference)
import jax
import jax.numpy as jnp
from jax import lax
from jax.experimental import pallas as pl
from jax.experimental.pallas import tpu as pltpu

N_DEV = 8
N_LANES = 8

PERM = (0, 1, 2, 3, 7, 6, 5, 4)


def kernel(x, w_mat):
    x = x.astype(jnp.bfloat16)
    w_mat = w_mat.astype(jnp.bfloat16)

    M, _ = x.shape
    _, N = w_mat.shape
    Mc = M // N_DEV
    W = N // N_LANES
    DIRS = [1 if li % 2 == 0 else -1 for li in range(N_LANES)]
    COL0 = [(li % 2) * (N // 2) + (li // 2) * W for li in range(N_LANES)]
    NSTEP = 2 * (N_DEV - 1)

    def body(x_ref, w_ref, out_ref, *scr):
        comms = scr[:N_LANES]
        send_sems, recv_sems, out_sems = scr[N_LANES : N_LANES + 3]
        credits = scr[N_LANES + 3 :]

        def perm(v):
            return jnp.where(v < 4, v, 11 - v)

        my_mesh = lax.axis_index("i")
        my = perm(my_mesh)
        tgt = [perm((my + d) % N_DEV) for d in DIRS]
        src = [perm((my - d) % N_DEV) for d in DIRS]

        barrier_sem = pltpu.get_barrier_semaphore()
        for nbr in (tgt[0], src[0]):
            pl.semaphore_signal(
                barrier_sem, inc=1,
                device_id=(nbr,), device_id_type=pl.DeviceIdType.MESH,
            )
        pl.semaphore_wait(barrier_sem, 2)

        def partial_chunk(c, li):
            xs = x_ref[pl.ds(c * Mc, Mc), :]
            ws = w_ref[:, COL0[li] : COL0[li] + W]
            return jnp.dot(
                xs, ws, preferred_element_type=jnp.float32
            ).astype(jnp.bfloat16)

        def make_rdma(li, g):
            ss, rs = g % 2, (g + 1) % 2
            return pltpu.make_async_remote_copy(
                src_ref=comms[li].at[ss],
                dst_ref=comms[li].at[rs],
                send_sem=send_sems.at[li, ss],
                recv_sem=recv_sems.at[li, rs],
                device_id=(tgt[li],),
                device_id_type=pl.DeviceIdType.MESH,
            )

        for li in range(N_LANES):
            comms[li][0, :, :] = partial_chunk(my, li)
        rdmas = [None] * N_LANES
        for li in range(N_LANES):
            rdmas[li] = make_rdma(li, 0)
            rdmas[li].start()
        ps = [partial_chunk((my - DIRS[li]) % N_DEV, li) for li in range(N_LANES)]
        pending = [None] * N_LANES

        for g in range(NSTEP):
            rs = (g + 1) % 2
            for li in range(N_LANES):
                rdmas[li].wait()
                if pending[li] is not None:
                    pending[li].wait()
                    pending[li] = None
                if g < N_DEV - 1:
                    acc = comms[li][rs, :, :] + ps[li]
                    if g == N_DEV - 2:
                        acc = jnp.maximum(acc, 0)
                    comms[li][rs, :, :] = acc
                    if g == N_DEV - 2:
                        own = (my + DIRS[li]) % N_DEV
                        cp = pltpu.make_async_copy(
                            comms[li].at[rs],
                            out_ref.at[pl.ds(own * Mc, Mc), pl.ds(COL0[li], W)],
                            out_sems.at[li],
                        )
                        cp.start()
                        pending[li] = cp
                else:
                    t = g - (N_DEV - 1)
                    idx = (my - DIRS[li] * t) % N_DEV
                    cp = pltpu.make_async_copy(
                        comms[li].at[rs],
                        out_ref.at[pl.ds(idx * Mc, Mc), pl.ds(COL0[li], W)],
                        out_sems.at[li],
                    )
                    cp.start()
                    pending[li] = cp
                if g < NSTEP - 1:
                    pl.semaphore_signal(
                        credits[li], inc=1,
                        device_id=(src[li],),
                        device_id_type=pl.DeviceIdType.MESH,
                    )
                if g + 1 < NSTEP:
                    pl.semaphore_wait(credits[li], 1)
                    rdmas[li] = make_rdma(li, g + 1)
                    rdmas[li].start()
                    if g + 1 < N_DEV - 1:
                        ps[li] = partial_chunk(
                            (my - DIRS[li] * (g + 2)) % N_DEV, li
                        )

        for li in range(N_LANES):
            if pending[li] is not None:
                pending[li].wait()

    scratch = (
        [pltpu.VMEM((2, Mc, W), jnp.bfloat16) for _ in range(N_LANES)]
        + [
            pltpu.SemaphoreType.DMA((N_LANES, 2)),
            pltpu.SemaphoreType.DMA((N_LANES, 2)),
            pltpu.SemaphoreType.DMA((N_LANES,)),
        ]
        + [pltpu.SemaphoreType.REGULAR for _ in range(N_LANES)]
    )

    return pl.pallas_call(
        body,
        out_shape=jax.ShapeDtypeStruct((M, N), jnp.bfloat16),
        in_specs=[
            pl.BlockSpec(memory_space=pltpu.VMEM),
            pl.BlockSpec(memory_space=pltpu.VMEM),
        ],
        out_specs=pl.BlockSpec(memory_space=pl.ANY),
        scratch_shapes=scratch,
        compiler_params=pltpu.CompilerParams(
            collective_id=0,
            vmem_limit_bytes=128 * 1024 * 1024,
        ),
    )(x, w_mat)


# device time: 558441 ns/iter; 1.2612x vs baseline; 1.2612x over previous
import jax
import jax.numpy as jnp
from jax import lax
from jax.experimental import pallas as pl
from jax.experimental.pallas import tpu as pltpu

N_DEV = 8
PASSES = 2
GW = (2560, 2816, 2816)
GCOL0 = (0, 2560, 5376)
GMASKS = ((1, 3, 4), (3, 4, 1), (4, 1, 3))
NG = 3
KROWS = (2048, 1024, 512, 512, 1024, 2048)


def kernel(x, w_mat):
    x = x.astype(jnp.bfloat16)
    w_mat = w_mat.astype(jnp.bfloat16)

    M, _ = x.shape
    _, N = w_mat.shape
    WP = tuple(w // PASSES for w in GW)

    def body(x_ref, w_ref, out_ref, *scr):
        sends = scr[0:NG]
        accs = scr[NG : 2 * NG]
        send_sems, recv_sems, out_sems = scr[2 * NG : 2 * NG + 3]

        m = lax.axis_index("i")
        cz = (m >> 2) & 1
        cy = (m >> 1) & 1
        cx = (m & 1) ^ cy
        bits = {1: cx, 3: cy, 4: cz}
        B = [tuple(bits[mask] for mask in GMASKS[g]) for g in range(NG)]
        Q = [tuple(m ^ mask for mask in GMASKS[g]) for g in range(NG)]

        barrier_sem = pltpu.get_barrier_semaphore()
        for mask in (1, 3, 4):
            pl.semaphore_signal(
                barrier_sem, inc=1,
                device_id=(m ^ mask,), device_id_type=pl.DeviceIdType.MESH,
            )
        pl.semaphore_wait(barrier_sem, 3)

        for p in range(PASSES):
            cols = [
                (GCOL0[g] + p * WP[g], GCOL0[g] + (p + 1) * WP[g])
                for g in range(NG)
            ]

            def partial(row0, nrows, g):
                xs = x_ref[pl.ds(row0, nrows), :]
                ws = w_ref[:, cols[g][0] : cols[g][1]]
                return jnp.dot(
                    xs, ws, preferred_element_type=jnp.float32
                ).astype(jnp.bfloat16)

            def exchange(g, k, src_ref, dst_ref):
                rdma = pltpu.make_async_remote_copy(
                    src_ref=src_ref,
                    dst_ref=dst_ref,
                    send_sem=send_sems.at[g, k],
                    recv_sem=recv_sems.at[g, k],
                    device_id=(Q[g][k if k < 3 else 5 - k],),
                    device_id_type=pl.DeviceIdType.MESH,
                )
                rdma.start()
                return rdma

            half = [B[g][0] * 2048 for g in range(NG)]
            q1_off = [B[g][1] * 1024 for g in range(NG)]
            q2_off = [
                B[g][1] * 1024 + B[g][2] * 512 for g in range(NG)
            ]
            out_cols = [pl.ds(cols[g][0], WP[g]) for g in range(NG)]

            rdmas = [None] * NG
            ocp = [None] * NG

            for g in range(NG):
                sends[g][pl.ds(0, 2048), :] = partial(
                    (1 - B[g][0]) * 2048, 2048, g
                )
                rdmas[g] = exchange(
                    g, 0, sends[g].at[pl.ds(0, 2048), :], accs[g].at[:, :]
                )

            for k in range(3):
                for g in range(NG):
                    b = B[g]
                    rdmas[g].wait()
                    if k == 0:
                        accs[g][:, :] = accs[g][:, :] + partial(
                            half[g], 2048, g
                        )
                        rdmas[g] = exchange(
                            g, 1,
                            accs[g].at[pl.ds((1 - b[1]) * 1024, 1024), :],
                            sends[g].at[pl.ds(2048, 1024), :],
                        )
                    elif k == 1:
                        accs[g][pl.ds(q1_off[g], 1024), :] = (
                            accs[g][pl.ds(q1_off[g], 1024), :]
                            + sends[g][pl.ds(2048, 1024), :]
                        )
                        rdmas[g] = exchange(
                            g, 2,
                            accs[g].at[pl.ds(q1_off[g] + (1 - b[2]) * 512, 512), :],
                            sends[g].at[pl.ds(3072, 512), :],
                        )
                    else:
                        own = accs[g][pl.ds(q2_off[g], 512), :] + sends[g][
                            pl.ds(3072, 512), :
                        ]
                        accs[g][pl.ds(q2_off[g], 512), :] = jnp.maximum(own, 0)
                        own_base = half[g] + q2_off[g]
                        cp = pltpu.make_async_copy(
                            accs[g].at[pl.ds(q2_off[g], 512), :],
                            out_ref.at[pl.ds(own_base, 512), out_cols[g]],
                            out_sems.at[g],
                        )
                        cp.start()
                        ocp[g] = cp
                        rdmas[g] = exchange(
                            g, 3,
                            accs[g].at[pl.ds(q2_off[g], 512), :],
                            out_ref.at[pl.ds(own_base, 512), out_cols[g]],
                        )

            for k in range(3, 6):
                for g in range(NG):
                    rdmas[g].wait()
                    if k == 3:
                        ocp[g].wait()
                        base = half[g] + q1_off[g]
                        rdmas[g] = exchange(
                            g, 4,
                            out_ref.at[pl.ds(base, 1024), out_cols[g]],
                            out_ref.at[pl.ds(base, 1024), out_cols[g]],
                        )
                    elif k == 4:
                        rdmas[g] = exchange(
                            g, 5,
                            out_ref.at[pl.ds(half[g], 2048), out_cols[g]],
                            out_ref.at[pl.ds(half[g], 2048), out_cols[g]],
                        )

    scratch = (
        [pltpu.VMEM((3584, WP[g]), jnp.bfloat16) for g in range(NG)]
        + [pltpu.VMEM((2048, WP[g]), jnp.bfloat16) for g in range(NG)]
        + [
            pltpu.SemaphoreType.DMA((NG, 6)),
            pltpu.SemaphoreType.DMA((NG, 6)),
            pltpu.SemaphoreType.DMA((NG,)),
        ]
    )

    return pl.pallas_call(
        body,
        out_shape=jax.ShapeDtypeStruct((M, N), jnp.bfloat16),
        in_specs=[
            pl.BlockSpec(memory_space=pltpu.VMEM),
            pl.BlockSpec(memory_space=pltpu.VMEM),
        ],
        out_specs=pl.BlockSpec(memory_space=pl.ANY),
        scratch_shapes=scratch,
        compiler_params=pltpu.CompilerParams(
            collective_id=0,
            vmem_limit_bytes=128 * 1024 * 1024,
        ),
    )(x, w_mat)


# device time: 544029 ns/iter; 1.2946x vs baseline; 1.0265x over previous
import jax
import jax.numpy as jnp
from jax import lax
from jax.experimental import pallas as pl
from jax.experimental.pallas import tpu as pltpu

N_DEV = 8
PASSES = 2
GW = (2560, 2816, 2816)
GCOL0 = (0, 2560, 5376)
GMASKS = ((1, 3, 4), (3, 4, 1), (4, 1, 3))
NG = 3
KROWS = (2048, 1024, 512, 512, 1024, 2048)


def kernel(x, w_mat):
    x = x.astype(jnp.bfloat16)
    w_mat = w_mat.astype(jnp.bfloat16)

    M, _ = x.shape
    _, N = w_mat.shape
    WP = tuple(w // PASSES for w in GW)

    def body(x_ref, w_ref, out_ref, *scr):
        sends = scr[0:NG]
        accs = scr[NG : 2 * NG]
        send_sems, recv_sems, out_sems = scr[2 * NG : 2 * NG + 3]
        hs = scr[2 * NG + 3 : 3 * NG + 3]
        hs1 = scr[3 * NG + 3 : 4 * NG + 3]
        hs2 = scr[4 * NG + 3 : 5 * NG + 3]

        m = lax.axis_index("i")
        cz = (m >> 2) & 1
        cy = (m >> 1) & 1
        cx = (m & 1) ^ cy
        bits = {1: cx, 3: cy, 4: cz}
        B = [tuple(bits[mask] for mask in GMASKS[g]) for g in range(NG)]
        Q = [tuple(m ^ mask for mask in GMASKS[g]) for g in range(NG)]

        barrier_sem = pltpu.get_barrier_semaphore()
        for mask in (1, 3, 4):
            pl.semaphore_signal(
                barrier_sem, inc=1,
                device_id=(m ^ mask,), device_id_type=pl.DeviceIdType.MESH,
            )
        pl.semaphore_wait(barrier_sem, 3)

        def col0(p, g):
            return GCOL0[g] + p * WP[g]

        def partial(p, row0, nrows, g):
            xs = x_ref[pl.ds(row0, nrows), :]
            ws = w_ref[:, col0(p, g) : col0(p, g) + WP[g]]
            return jnp.dot(
                xs, ws, preferred_element_type=jnp.float32
            ).astype(jnp.bfloat16)

        def exchange(g, k, src_ref, dst_ref):
            rdma = pltpu.make_async_remote_copy(
                src_ref=src_ref,
                dst_ref=dst_ref,
                send_sem=send_sems.at[g, k],
                recv_sem=recv_sems.at[g, k],
                device_id=(Q[g][k if k < 3 else 5 - k],),
                device_id_type=pl.DeviceIdType.MESH,
            )
            rdma.start()
            return rdma

        def fire_k0(p, g):
            sends[g][pl.ds(0, 2048), :] = partial(
                p, (1 - B[g][0]) * 2048, 2048, g
            )
            return exchange(
                g, 0, sends[g].at[pl.ds(0, 2048), :], accs[g].at[:, :]
            )

        half = [B[g][0] * 2048 for g in range(NG)]
        q1_off = [B[g][1] * 1024 for g in range(NG)]
        q2_off = [
            B[g][1] * 1024 + B[g][2] * 512 for g in range(NG)
        ]

        rdmas = [fire_k0(0, g) for g in range(NG)]
        nxt = [None] * NG
        ocp = [None] * NG

        for p in range(PASSES):
            out_cols = [pl.ds(col0(p, g), WP[g]) for g in range(NG)]

            for k in range(3):
                for g in range(NG):
                    b = B[g]
                    rdmas[g].wait()
                    if k == 0:
                        accs[g][:, :] = accs[g][:, :] + partial(
                            p, half[g], 2048, g
                        )
                        if p > 0:
                            pl.semaphore_wait(hs1[g], 1)
                        rdmas[g] = exchange(
                            g, 1,
                            accs[g].at[pl.ds((1 - b[1]) * 1024, 1024), :],
                            sends[g].at[pl.ds(2048, 1024), :],
                        )
                    elif k == 1:
                        accs[g][pl.ds(q1_off[g], 1024), :] = (
                            accs[g][pl.ds(q1_off[g], 1024), :]
                            + sends[g][pl.ds(2048, 1024), :]
                        )
                        if p + 1 < PASSES:
                            pl.semaphore_signal(
                                hs1[g], inc=1,
                                device_id=(Q[g][1],),
                                device_id_type=pl.DeviceIdType.MESH,
                            )
                        if p > 0:
                            pl.semaphore_wait(hs2[g], 1)
                        rdmas[g] = exchange(
                            g, 2,
                            accs[g].at[pl.ds(q1_off[g] + (1 - b[2]) * 512, 512), :],
                            sends[g].at[pl.ds(3072, 512), :],
                        )
                    else:
                        own = accs[g][pl.ds(q2_off[g], 512), :] + sends[g][
                            pl.ds(3072, 512), :
                        ]
                        accs[g][pl.ds(q2_off[g], 512), :] = jnp.maximum(own, 0)
                        if p + 1 < PASSES:
                            pl.semaphore_signal(
                                hs2[g], inc=1,
                                device_id=(Q[g][2],),
                                device_id_type=pl.DeviceIdType.MESH,
                            )
                        own_base = half[g] + q2_off[g]
                        cp = pltpu.make_async_copy(
                            accs[g].at[pl.ds(q2_off[g], 512), :],
                            out_ref.at[pl.ds(own_base, 512), out_cols[g]],
                            out_sems.at[g],
                        )
                        cp.start()
                        ocp[g] = cp
                        rdmas[g] = exchange(
                            g, 3,
                            accs[g].at[pl.ds(q2_off[g], 512), :],
                            out_ref.at[pl.ds(own_base, 512), out_cols[g]],
                        )

            for k in range(3, 6):
                for g in range(NG):
                    rdmas[g].wait()
                    if k == 3:
                        ocp[g].wait()
                        if p + 1 < PASSES:
                            pl.semaphore_signal(
                                hs[g], inc=1,
                                device_id=(Q[g][0],),
                                device_id_type=pl.DeviceIdType.MESH,
                            )
                        base = half[g] + q1_off[g]
                        rdmas[g] = exchange(
                            g, 4,
                            out_ref.at[pl.ds(base, 1024), out_cols[g]],
                            out_ref.at[pl.ds(base, 1024), out_cols[g]],
                        )
                        if p + 1 < PASSES:
                            pl.semaphore_wait(hs[g], 1)
                            nxt[g] = fire_k0(p + 1, g)
                    elif k == 4:
                        rdmas[g] = exchange(
                            g, 5,
                            out_ref.at[pl.ds(half[g], 2048), out_cols[g]],
                            out_ref.at[pl.ds(half[g], 2048), out_cols[g]],
                        )

            if p + 1 < PASSES:
                rdmas, nxt = nxt, rdmas

    scratch = (
        [pltpu.VMEM((3584, WP[g]), jnp.bfloat16) for g in range(NG)]
        + [pltpu.VMEM((2048, WP[g]), jnp.bfloat16) for g in range(NG)]
        + [
            pltpu.SemaphoreType.DMA((NG, 6)),
            pltpu.SemaphoreType.DMA((NG, 6)),
            pltpu.SemaphoreType.DMA((NG,)),
        ]
        + [pltpu.SemaphoreType.REGULAR for _ in range(3 * NG)]
    )

    return pl.pallas_call(
        body,
        out_shape=jax.ShapeDtypeStruct((M, N), jnp.bfloat16),
        in_specs=[
            pl.BlockSpec(memory_space=pltpu.VMEM),
            pl.BlockSpec(memory_space=pltpu.VMEM),
        ],
        out_specs=pl.BlockSpec(memory_space=pl.ANY),
        scratch_shapes=scratch,
        compiler_params=pltpu.CompilerParams(
            collective_id=0,
            vmem_limit_bytes=128 * 1024 * 1024,
        ),
    )(x, w_mat)


# device time: 537315 ns/iter; 1.3108x vs baseline; 1.0125x over previous
import jax
import jax.numpy as jnp
from jax import lax
from jax.experimental import pallas as pl
from jax.experimental.pallas import tpu as pltpu

N_DEV = 8
PASSES = 2
GW = (2560, 2816, 2816)
GCOL0 = (0, 2560, 5376)
GMASKS = ((1, 3, 4), (3, 4, 1), (4, 1, 3))
NG = 3
KROWS = (2048, 1024, 512, 512, 1024, 2048)


def kernel(x, w_mat):
    x = x.astype(jnp.bfloat16)
    w_mat = w_mat.astype(jnp.bfloat16)

    M, _ = x.shape
    _, N = w_mat.shape
    WP = tuple(w // PASSES for w in GW)

    def body(x_ref, w_ref, out_ref, *scr):
        sends = scr[0:NG]
        accs = scr[NG : 2 * NG]
        send_sems, recv_sems, out_sems = scr[2 * NG : 2 * NG + 3]
        hs = scr[2 * NG + 3 : 3 * NG + 3]
        hs1 = scr[3 * NG + 3 : 4 * NG + 3]
        hs2 = scr[4 * NG + 3 : 5 * NG + 3]

        m = lax.axis_index("i")
        cz = (m >> 2) & 1
        cy = (m >> 1) & 1
        cx = (m & 1) ^ cy
        bits = {1: cx, 3: cy, 4: cz}
        B = [tuple(bits[mask] for mask in GMASKS[g]) for g in range(NG)]
        Q = [tuple(m ^ mask for mask in GMASKS[g]) for g in range(NG)]

        barrier_sem = pltpu.get_barrier_semaphore()
        for mask in (1, 3, 4):
            pl.semaphore_signal(
                barrier_sem, inc=1,
                device_id=(m ^ mask,), device_id_type=pl.DeviceIdType.MESH,
            )
        pl.semaphore_wait(barrier_sem, 3)

        def col0(p, g):
            return GCOL0[g] + p * WP[g]

        def partial(p, row0, nrows, g):
            xs = x_ref[pl.ds(row0, nrows), :]
            ws = w_ref[:, col0(p, g) : col0(p, g) + WP[g]]
            return jnp.dot(
                xs, ws, preferred_element_type=jnp.float32
            ).astype(jnp.bfloat16)

        def exchange(g, k, src_ref, dst_ref, slot=None):
            rdma = pltpu.make_async_remote_copy(
                src_ref=src_ref,
                dst_ref=dst_ref,
                send_sem=send_sems.at[g, k if slot is None else slot],
                recv_sem=recv_sems.at[g, k if slot is None else slot],
                device_id=(Q[g][k if k < 3 else 5 - k],),
                device_id_type=pl.DeviceIdType.MESH,
            )
            rdma.start()
            return rdma

        def fire_k0(p, g):
            b1 = B[g][1]
            sends[g][pl.ds(0, 2048), :] = partial(
                p, (1 - B[g][0]) * 2048, 2048, g
            )
            sub_a = exchange(
                g, 0,
                sends[g].at[pl.ds((1 - b1) * 1024, 1024), :],
                accs[g].at[pl.ds((1 - b1) * 1024, 1024), :],
            )
            sub_b = exchange(
                g, 0,
                sends[g].at[pl.ds(b1 * 1024, 1024), :],
                accs[g].at[pl.ds(b1 * 1024, 1024), :],
                slot=6,
            )
            return (sub_a, sub_b)

        half = [B[g][0] * 2048 for g in range(NG)]
        q1_off = [B[g][1] * 1024 for g in range(NG)]
        q2_off = [
            B[g][1] * 1024 + B[g][2] * 512 for g in range(NG)
        ]

        rdmas = [fire_k0(0, g) for g in range(NG)]
        nxt = [None] * NG
        ocp = [None] * NG

        for p in range(PASSES):
            out_cols = [pl.ds(col0(p, g), WP[g]) for g in range(NG)]

            for k in range(3):
                for g in range(NG):
                    b = B[g]
                    if k == 0:
                        sub_a, sub_b = rdmas[g]
                        sub_a.wait()
                        accs[g][pl.ds((1 - b[1]) * 1024, 1024), :] = accs[g][
                            pl.ds((1 - b[1]) * 1024, 1024), :
                        ] + partial(p, half[g] + (1 - b[1]) * 1024, 1024, g)
                        if p > 0:
                            pl.semaphore_wait(hs1[g], 1)
                        rdmas[g] = exchange(
                            g, 1,
                            accs[g].at[pl.ds((1 - b[1]) * 1024, 1024), :],
                            sends[g].at[pl.ds(2048, 1024), :],
                        )
                        sub_b.wait()
                        accs[g][pl.ds(b[1] * 1024, 1024), :] = accs[g][
                            pl.ds(b[1] * 1024, 1024), :
                        ] + partial(p, half[g] + b[1] * 1024, 1024, g)
                    elif k == 1:
                        rdmas[g].wait()
                        accs[g][pl.ds(q1_off[g], 1024), :] = (
                            accs[g][pl.ds(q1_off[g], 1024), :]
                            + sends[g][pl.ds(2048, 1024), :]
                        )
                        if p + 1 < PASSES:
                            pl.semaphore_signal(
                                hs1[g], inc=1,
                                device_id=(Q[g][1],),
                                device_id_type=pl.DeviceIdType.MESH,
                            )
                        if p > 0:
                            pl.semaphore_wait(hs2[g], 1)
                        rdmas[g] = exchange(
                            g, 2,
                            accs[g].at[pl.ds(q1_off[g] + (1 - b[2]) * 512, 512), :],
                            sends[g].at[pl.ds(3072, 512), :],
                        )
                    else:
                        rdmas[g].wait()
                        own = accs[g][pl.ds(q2_off[g], 512), :] + sends[g][
                            pl.ds(3072, 512), :
                        ]
                        accs[g][pl.ds(q2_off[g], 512), :] = jnp.maximum(own, 0)
                        if p + 1 < PASSES:
                            pl.semaphore_signal(
                                hs2[g], inc=1,
                                device_id=(Q[g][2],),
                                device_id_type=pl.DeviceIdType.MESH,
                            )
                        own_base = half[g] + q2_off[g]
                        cp = pltpu.make_async_copy(
                            accs[g].at[pl.ds(q2_off[g], 512), :],
                            out_ref.at[pl.ds(own_base, 512), out_cols[g]],
                            out_sems.at[g],
                        )
                        cp.start()
                        ocp[g] = cp
                        rdmas[g] = exchange(
                            g, 3,
                            accs[g].at[pl.ds(q2_off[g], 512), :],
                            out_ref.at[pl.ds(own_base, 512), out_cols[g]],
                        )

            for k in range(3, 6):
                for g in range(NG):
                    rdmas[g].wait()
                    if k == 3:
                        ocp[g].wait()
                        if p + 1 < PASSES:
                            pl.semaphore_signal(
                                hs[g], inc=1,
                                device_id=(Q[g][0],),
                                device_id_type=pl.DeviceIdType.MESH,
                            )
                        base = half[g] + q1_off[g]
                        rdmas[g] = exchange(
                            g, 4,
                            out_ref.at[pl.ds(base, 1024), out_cols[g]],
                            out_ref.at[pl.ds(base, 1024), out_cols[g]],
                        )
                        if p + 1 < PASSES:
                            pl.semaphore_wait(hs[g], 1)
                            nxt[g] = fire_k0(p + 1, g)
                    elif k == 4:
                        rdmas[g] = exchange(
                            g, 5,
                            out_ref.at[pl.ds(half[g], 2048), out_cols[g]],
                            out_ref.at[pl.ds(half[g], 2048), out_cols[g]],
                        )

            if p + 1 < PASSES:
                rdmas, nxt = nxt, rdmas

    scratch = (
        [pltpu.VMEM((3584, WP[g]), jnp.bfloat16) for g in range(NG)]
        + [pltpu.VMEM((2048, WP[g]), jnp.bfloat16) for g in range(NG)]
        + [
            pltpu.SemaphoreType.DMA((NG, 7)),
            pltpu.SemaphoreType.DMA((NG, 7)),
            pltpu.SemaphoreType.DMA((NG,)),
        ]
        + [pltpu.SemaphoreType.REGULAR for _ in range(3 * NG)]
    )

    return pl.pallas_call(
        body,
        out_shape=jax.ShapeDtypeStruct((M, N), jnp.bfloat16),
        in_specs=[
            pl.BlockSpec(memory_space=pltpu.VMEM),
            pl.BlockSpec(memory_space=pltpu.VMEM),
        ],
        out_specs=pl.BlockSpec(memory_space=pl.ANY),
        scratch_shapes=scratch,
        compiler_params=pltpu.CompilerParams(
            collective_id=0,
            vmem_limit_bytes=128 * 1024 * 1024,
        ),
    )(x, w_mat)


# device time: 535347 ns/iter; 1.3156x vs baseline; 1.0037x over previous
import jax
import jax.numpy as jnp
from jax import lax
from jax.experimental import pallas as pl
from jax.experimental.pallas import tpu as pltpu

N_DEV = 8
PASSES = 2
GW = (2560, 2816, 2816)
GCOL0 = (0, 2560, 5376)
GMASKS = ((1, 3, 4), (3, 4, 1), (4, 1, 3))
NG = 3
KROWS = (2048, 1024, 512, 512, 1024, 2048)


def kernel(x, w_mat):
    x = x.astype(jnp.bfloat16)
    w_mat = w_mat.astype(jnp.bfloat16)

    M, _ = x.shape
    _, N = w_mat.shape
    WP = tuple(w // PASSES for w in GW)

    def body(x_ref, w_ref, out_ref, *scr):
        sends = scr[0:NG]
        accs = scr[NG : 2 * NG]
        send_sems, recv_sems, out_sems = scr[2 * NG : 2 * NG + 3]
        hs = scr[2 * NG + 3 : 3 * NG + 3]
        hs1 = scr[3 * NG + 3 : 4 * NG + 3]
        hs2 = scr[4 * NG + 3 : 5 * NG + 3]
        hs0 = scr[5 * NG + 3 : 6 * NG + 3]

        m = lax.axis_index("i")
        cz = (m >> 2) & 1
        cy = (m >> 1) & 1
        cx = (m & 1) ^ cy
        bits = {1: cx, 3: cy, 4: cz}
        B = [tuple(bits[mask] for mask in GMASKS[g]) for g in range(NG)]
        Q = [tuple(m ^ mask for mask in GMASKS[g]) for g in range(NG)]

        barrier_sem = pltpu.get_barrier_semaphore()
        for mask in (1, 3, 4):
            pl.semaphore_signal(
                barrier_sem, inc=1,
                device_id=(m ^ mask,), device_id_type=pl.DeviceIdType.MESH,
            )
        pl.semaphore_wait(barrier_sem, 3)

        def col0(p, g):
            return GCOL0[g] + p * WP[g]

        def partial(p, row0, nrows, g):
            xs = x_ref[pl.ds(row0, nrows), :]
            ws = w_ref[:, col0(p, g) : col0(p, g) + WP[g]]
            return jnp.dot(
                xs, ws, preferred_element_type=jnp.float32
            ).astype(jnp.bfloat16)

        def exchange(g, k, src_ref, dst_ref, slot=None):
            rdma = pltpu.make_async_remote_copy(
                src_ref=src_ref,
                dst_ref=dst_ref,
                send_sem=send_sems.at[g, k if slot is None else slot],
                recv_sem=recv_sems.at[g, k if slot is None else slot],
                device_id=(Q[g][k if k < 3 else 5 - k],),
                device_id_type=pl.DeviceIdType.MESH,
            )
            rdma.start()
            return rdma

        def fire_k0(p, g):
            b1 = B[g][1]
            sends[g][pl.ds(0, 2048), :] = partial(
                p, (1 - B[g][0]) * 2048, 2048, g
            )
            sub_a = exchange(
                g, 0,
                sends[g].at[pl.ds((1 - b1) * 1024, 1024), :],
                accs[g].at[pl.ds((1 - b1) * 1024, 1024), :],
            )
            sub_b = exchange(
                g, 0,
                sends[g].at[pl.ds(b1 * 1024, 1024), :],
                accs[g].at[pl.ds(b1 * 1024, 1024), :],
                slot=6,
            )
            return (sub_a, sub_b)

        half = [B[g][0] * 2048 for g in range(NG)]
        q1_off = [B[g][1] * 1024 for g in range(NG)]
        q2_off = [
            B[g][1] * 1024 + B[g][2] * 512 for g in range(NG)
        ]

        rdmas = [fire_k0(0, g) for g in range(NG)]
        nxt_a = [None] * NG
        nxt_b = [None] * NG
        ocp = [None] * NG

        for p in range(PASSES):
            out_cols = [pl.ds(col0(p, g), WP[g]) for g in range(NG)]

            for k in range(3):
                for g in range(NG):
                    b = B[g]
                    if k == 0:
                        sub_a, sub_b = rdmas[g]
                        sub_a.wait()
                        accs[g][pl.ds((1 - b[1]) * 1024, 1024), :] = accs[g][
                            pl.ds((1 - b[1]) * 1024, 1024), :
                        ] + partial(p, half[g] + (1 - b[1]) * 1024, 1024, g)
                        if p > 0:
                            pl.semaphore_wait(hs1[g], 1)
                        rdmas[g] = exchange(
                            g, 1,
                            accs[g].at[pl.ds((1 - b[1]) * 1024, 1024), :],
                            sends[g].at[pl.ds(2048, 1024), :],
                        )
                        sub_b.wait()
                        accs[g][pl.ds(b[1] * 1024, 1024), :] = accs[g][
                            pl.ds(b[1] * 1024, 1024), :
                        ] + partial(p, half[g] + b[1] * 1024, 1024, g)
                    elif k == 1:
                        rdmas[g].wait()
                        if p + 1 < PASSES:
                            pl.semaphore_signal(
                                hs0[g], inc=1,
                                device_id=(Q[g][0],),
                                device_id_type=pl.DeviceIdType.MESH,
                            )
                        accs[g][pl.ds(q1_off[g], 1024), :] = (
                            accs[g][pl.ds(q1_off[g], 1024), :]
                            + sends[g][pl.ds(2048, 1024), :]
                        )
                        if p + 1 < PASSES:
                            pl.semaphore_signal(
                                hs1[g], inc=1,
                                device_id=(Q[g][1],),
                                device_id_type=pl.DeviceIdType.MESH,
                            )
                        if p > 0:
                            pl.semaphore_wait(hs2[g], 1)
                        rdmas[g] = exchange(
                            g, 2,
                            accs[g].at[pl.ds(q1_off[g] + (1 - b[2]) * 512, 512), :],
                            sends[g].at[pl.ds(3072, 512), :],
                        )
                        if p + 1 < PASSES:
                            sends[g][pl.ds((1 - b[1]) * 1024, 1024), :] = (
                                partial(
                                    p + 1,
                                    (1 - b[0]) * 2048 + (1 - b[1]) * 1024,
                                    1024,
                                    g,
                                )
                            )
                            pl.semaphore_wait(hs0[g], 1)
                            nxt_a[g] = exchange(
                                g, 0,
                                sends[g].at[pl.ds((1 - b[1]) * 1024, 1024), :],
                                accs[g].at[pl.ds((1 - b[1]) * 1024, 1024), :],
                            )
                    else:
                        rdmas[g].wait()
                        own = accs[g][pl.ds(q2_off[g], 512), :] + sends[g][
                            pl.ds(3072, 512), :
                        ]
                        accs[g][pl.ds(q2_off[g], 512), :] = jnp.maximum(own, 0)
                        if p + 1 < PASSES:
                            pl.semaphore_signal(
                                hs2[g], inc=1,
                                device_id=(Q[g][2],),
                                device_id_type=pl.DeviceIdType.MESH,
                            )
                        own_base = half[g] + q2_off[g]
                        cp = pltpu.make_async_copy(
                            accs[g].at[pl.ds(q2_off[g], 512), :],
                            out_ref.at[pl.ds(own_base, 512), out_cols[g]],
                            out_sems.at[g],
                        )
                        cp.start()
                        ocp[g] = cp
                        rdmas[g] = exchange(
                            g, 3,
                            accs[g].at[pl.ds(q2_off[g], 512), :],
                            out_ref.at[pl.ds(own_base, 512), out_cols[g]],
                        )

            for k in range(3, 6):
                for g in range(NG):
                    rdmas[g].wait()
                    if k == 3:
                        ocp[g].wait()
                        if p + 1 < PASSES:
                            pl.semaphore_signal(
                                hs[g], inc=1,
                                device_id=(Q[g][0],),
                                device_id_type=pl.DeviceIdType.MESH,
                            )
                        base = half[g] + q1_off[g]
                        rdmas[g] = exchange(
                            g, 4,
                            out_ref.at[pl.ds(base, 1024), out_cols[g]],
                            out_ref.at[pl.ds(base, 1024), out_cols[g]],
                        )
                        if p + 1 < PASSES:
                            b = B[g]
                            sends[g][pl.ds(b[1] * 1024, 1024), :] = partial(
                                p + 1,
                                (1 - b[0]) * 2048 + b[1] * 1024,
                                1024,
                                g,
                            )
                            pl.semaphore_wait(hs[g], 1)
                            nxt_b[g] = exchange(
                                g, 0,
                                sends[g].at[pl.ds(b[1] * 1024, 1024), :],
                                accs[g].at[pl.ds(b[1] * 1024, 1024), :],
                                slot=6,
                            )
                    elif k == 4:
                        rdmas[g] = exchange(
                            g, 5,
                            out_ref.at[pl.ds(half[g], 2048), out_cols[g]],
                            out_ref.at[pl.ds(half[g], 2048), out_cols[g]],
                        )

            if p + 1 < PASSES:
                rdmas = [(nxt_a[g], nxt_b[g]) for g in range(NG)]

    scratch = (
        [pltpu.VMEM((3584, WP[g]), jnp.bfloat16) for g in range(NG)]
        + [pltpu.VMEM((2048, WP[g]), jnp.bfloat16) for g in range(NG)]
        + [
            pltpu.SemaphoreType.DMA((NG, 7)),
            pltpu.SemaphoreType.DMA((NG, 7)),
            pltpu.SemaphoreType.DMA((NG,)),
        ]
        + [pltpu.SemaphoreType.REGULAR for _ in range(4 * NG)]
    )

    return pl.pallas_call(
        body,
        out_shape=jax.ShapeDtypeStruct((M, N), jnp.bfloat16),
        in_specs=[
            pl.BlockSpec(memory_space=pltpu.VMEM),
            pl.BlockSpec(memory_space=pltpu.VMEM),
        ],
        out_specs=pl.BlockSpec(memory_space=pl.ANY),
        scratch_shapes=scratch,
        compiler_params=pltpu.CompilerParams(
            collective_id=0,
            vmem_limit_bytes=128 * 1024 * 1024,
        ),
    )(x, w_mat)
